# Initial kernel scaffold; baseline (speedup 1.0000x reference)
import jax
import jax.numpy as jnp
from jax.experimental import pallas as pl


def kernel(x, edge_index, W1, b1, W2, b2, Wfc, bfc):
    raise NotImplementedError("write your pallas kernel here")


# speedup vs baseline: 8.6873x; 8.6873x over previous
import functools

import jax
import jax.numpy as jnp
from jax import lax
from jax.experimental import pallas as pl
from jax.experimental.pallas import tpu as pltpu
from jax.experimental.pallas import tpu_sc as plsc

N = 10000
E = 320000
D = 128
NC = 2
NS = 16
NW = NC * NS
K = 128
S = 80
EPAD = NW * S * K
ACC_ROWS = 10240
ZR = ACC_ROWS // NS
BM = 1000


_SC_MESH = plsc.VectorSubcoreMesh(core_axis_name="c", subcore_axis_name="s")


def _deg_body(dst_hbm, out_hbm, dst_v, ones_v, zbuf, acc, sem):
    c = lax.axis_index("c")
    s = lax.axis_index("s")
    wid = c * NS + s

    one16 = jnp.full((16,), 1.0, jnp.float32)
    zero16 = jnp.zeros((16,), jnp.float32)
    for i in range(16):
        zbuf[i, pl.ds(0, 16)] = zero16
    for i in range(K):
        ones_v[i, pl.ds(0, 16)] = one16

    base = s * ZR
    for t in range(ZR // 16):
        pltpu.sync_copy(zbuf, acc.at[pl.ds(base + t * 16, 16)])
    pltpu.sync_copy(dst_hbm.at[wid], dst_v)
    plsc.subcore_barrier()

    def step(j, carry):
        pltpu.sync_copy(ones_v, acc.at[dst_v.at[j]], add=True)
        return carry

    lax.fori_loop(0, S, step, 0)
    plsc.subcore_barrier()

    pltpu.sync_copy(acc.at[pl.ds(base, ZR)], out_hbm.at[c].at[pl.ds(base, ZR)])


_deg_call = pl.kernel(
    _deg_body,
    out_type=jax.ShapeDtypeStruct((NC, ACC_ROWS, 16), jnp.float32),
    mesh=_SC_MESH,
    scratch_types=[
        pltpu.VMEM((S, K), jnp.int32),
        pltpu.VMEM((K, 16), jnp.float32),
        pltpu.VMEM((16, 16), jnp.float32),
        pltpu.VMEM_SHARED((ACC_ROWS, 16), jnp.float32),
        pltpu.SemaphoreType.DMA,
    ],
    name="sc_deg_count",
)


def _scatter_body(h_hbm, src_hbm, dst_hbm, out_hbm,
                  src_v, dst_v, rows_v, zbuf, acc, sem):
    c = lax.axis_index("c")
    s = lax.axis_index("s")
    wid = c * NS + s

    zero16 = jnp.zeros((16,), jnp.float32)
    for i in range(16):
        for jj in range(8):
            zbuf[i, pl.ds(jj * 16, 16)] = zero16

    base = s * ZR
    for t in range(ZR // 16):
        pltpu.sync_copy(zbuf, acc.at[pl.ds(base + t * 16, 16)])
    pltpu.sync_copy(src_hbm.at[wid], src_v)
    pltpu.sync_copy(dst_hbm.at[wid], dst_v)
    plsc.subcore_barrier()

    def step(j, carry):
        pltpu.async_copy(h_hbm.at[src_v.at[j]], rows_v, sem).wait()
        pltpu.sync_copy(rows_v, acc.at[dst_v.at[j]], add=True)
        return carry

    lax.fori_loop(0, S, step, 0)
    plsc.subcore_barrier()

    pltpu.sync_copy(acc.at[pl.ds(base, ZR)], out_hbm.at[c].at[pl.ds(base, ZR)])


_scatter_call = pl.kernel(
    _scatter_body,
    out_type=jax.ShapeDtypeStruct((NC, ACC_ROWS, D), jnp.float32),
    mesh=_SC_MESH,
    scratch_types=[
        pltpu.VMEM((S, K), jnp.int32),
        pltpu.VMEM((S, K), jnp.int32),
        pltpu.VMEM((K, D), jnp.float32),
        pltpu.VMEM((16, D), jnp.float32),
        pltpu.VMEM_SHARED((ACC_ROWS, D), jnp.float32),
        pltpu.SemaphoreType.DMA,
    ],
    name="sc_edge_scatter",
)


def _dis_block(degp_ref):
    return lax.rsqrt(1.0 + degp_ref[0, :, 0:1] + degp_ref[1, :, 0:1])


def _mm1_body(x_ref, w_ref, degp_ref, o_ref):
    h = jnp.dot(x_ref[...], w_ref[...], preferred_element_type=jnp.float32)
    o_ref[...] = h * _dis_block(degp_ref)


def _mm2_body(a_ref, hp_ref, degp_ref, w_ref, b_ref, o_ref):
    dis = _dis_block(degp_ref)
    pre = dis * (a_ref[0] + a_ref[1] + hp_ref[...]) + b_ref[...]
    act = jnp.where(pre > 0, pre, jnp.exp(pre) - 1.0)
    h = jnp.dot(act, w_ref[...], preferred_element_type=jnp.float32)
    o_ref[...] = h * dis


def _mm3_body(a_ref, hp_ref, degp_ref, w_ref, b_ref, bfc_ref, o_ref):
    dis = _dis_block(degp_ref)
    pre = dis * (a_ref[0] + a_ref[1] + hp_ref[...]) + b_ref[...]
    act = jnp.where(pre > 0, pre, jnp.exp(pre) - 1.0)
    logits = jnp.dot(act, w_ref[...], preferred_element_type=jnp.float32)
    logits = logits + bfc_ref[...]
    m = jnp.max(logits, axis=1, keepdims=True)
    e = jnp.exp(logits - m)
    o_ref[...] = e / jnp.sum(e, axis=1, keepdims=True)


_bs_rows = pl.BlockSpec((BM, D), lambda i: (i, 0))
_bs_w = pl.BlockSpec((D, D), lambda i: (0, 0))
_bs_b = pl.BlockSpec((1, D), lambda i: (0, 0))
_bs_degp = pl.BlockSpec((NC, BM, 16), lambda i: (0, i, 0))
_bs_parts = pl.BlockSpec((NC, BM, D), lambda i: (0, i, 0))
_GRID = (N // BM,)

_mm1 = pl.pallas_call(
    _mm1_body,
    grid=_GRID,
    in_specs=[_bs_rows, _bs_w, _bs_degp],
    out_specs=_bs_rows,
    out_shape=jax.ShapeDtypeStruct((N, D), jnp.float32),
)

_mm2 = pl.pallas_call(
    _mm2_body,
    grid=_GRID,
    in_specs=[_bs_parts, _bs_rows, _bs_degp, _bs_w, _bs_b],
    out_specs=_bs_rows,
    out_shape=jax.ShapeDtypeStruct((N, D), jnp.float32),
)

_mm3 = pl.pallas_call(
    _mm3_body,
    grid=_GRID,
    in_specs=[_bs_parts, _bs_rows, _bs_degp, _bs_w, _bs_b, _bs_b],
    out_specs=_bs_rows,
    out_shape=jax.ShapeDtypeStruct((N, D), jnp.float32),
)


def kernel(x, edge_index, W1, b1, W2, b2, Wfc, bfc):
    src = edge_index[0].astype(jnp.int32)
    dst = edge_index[1].astype(jnp.int32)
    pad = EPAD - E
    src_p = jnp.concatenate([src, jnp.zeros((pad,), jnp.int32)]).reshape(NW, S, K)
    dst_p = jnp.concatenate([dst, jnp.full((pad,), N, jnp.int32)]).reshape(NW, S, K)

    degp = _deg_call(dst_p)
    h1p = _mm1(x, W1, degp)
    a1 = _scatter_call(h1p, src_p, dst_p)
    h2p = _mm2(a1, h1p, degp, W2, b1.reshape(1, D))
    a2 = _scatter_call(h2p, src_p, dst_p)
    out = _mm3(a2, h2p, degp, Wfc, b2.reshape(1, D), bfc.reshape(1, D))
    return out
